# baseline (speedup 1.0000x reference)
import functools

import jax
import jax.numpy as jnp
from jax import lax
from jax.experimental import pallas as pl
from jax.experimental.pallas import tpu as pltpu
from jax.experimental.pallas import tpu_sc as plsc

N = 1 << 20
N_ACTIONS = 4
NC = 2
NS = 16
L = 16
SEG = N // N_ACTIONS
W_PER_SEG = (NC * NS) // N_ACTIONS
CHUNK = SEG // W_PER_SEG
NBUF = 4
SUB = CHUNK // NBUF
ACCS = 8
STEPS = SUB // (ACCS * L)


def _policy_body(probs_hbm, out_hbm, chunk_v, acc_v, part_v, partials_hbm, *sems):
    c = lax.axis_index("c")
    sid = lax.axis_index("s")
    seg = c * (N_ACTIONS // NC) + sid // W_PER_SEG
    base = seg * SEG + (sid % W_PER_SEG) * CHUNK

    copies = [
        pltpu.async_copy(
            probs_hbm.at[pl.ds(base + b * SUB, SUB)], chunk_v.at[b], sems[b]
        )
        for b in range(NBUF)
    ]

    zero = jnp.zeros((L,), jnp.float32)
    accs = (zero,) * ACCS
    for b in range(NBUF):
        copies[b].wait()

        def body(i, accs, b=b):
            off = i * (2 * ACCS * L)
            accs = tuple(
                a + chunk_v[b, pl.ds(off + k * L, L)]
                for k, a in enumerate(accs)
            )
            return tuple(
                a + chunk_v[b, pl.ds(off + (ACCS + k) * L, L)]
                for k, a in enumerate(accs)
            )

        accs = lax.fori_loop(0, STEPS // 2, body, accs)
    acc = accs[0]
    for a in accs[1:]:
        acc = acc + a
    acc_v[...] = acc
    pltpu.sync_copy(acc_v, partials_hbm.at[c, sid])
    plsc.subcore_barrier()

    @pl.when(sid % W_PER_SEG == 0)
    def _():
        pltpu.sync_copy(partials_hbm.at[c, pl.ds(sid, W_PER_SEG)], part_v)
        tot = part_v[0]
        for k in range(1, W_PER_SEG):
            tot = tot + part_v[k]
        total = tot[0]
        for k in range(1, L):
            total = total + tot[k]
        lane = lax.iota(jnp.int32, L)
        acc_v[...] = jnp.where(lane == 0, total, jnp.float32(0.0))
        pltpu.sync_copy(acc_v, out_hbm.at[seg])


_policy_sc = functools.partial(
    pl.kernel,
    out_type=jax.ShapeDtypeStruct((N_ACTIONS, L), jnp.float32),
    mesh=plsc.VectorSubcoreMesh(
        core_axis_name="c", subcore_axis_name="s", num_cores=NC, num_subcores=NS
    ),
    scratch_types=[
        pltpu.VMEM((NBUF, SUB), jnp.float32),
        pltpu.VMEM((L,), jnp.float32),
        pltpu.VMEM((W_PER_SEG, L), jnp.float32),
        pltpu.HBM((NC, NS, L), jnp.float32),
    ] + [pltpu.SemaphoreType.DMA] * NBUF,
)(_policy_body)


def kernel(probs):
    out16 = _policy_sc(probs.reshape(-1))
    return out16[:, 0]


# speedup vs baseline: 1.0443x; 1.0443x over previous
import functools

import jax
import jax.numpy as jnp
from jax import lax
from jax.experimental import pallas as pl
from jax.experimental.pallas import tpu as pltpu
from jax.experimental.pallas import tpu_sc as plsc

N = 1 << 20
N_ACTIONS = 4
NC = 2
NS = 16
L = 16
SEG = N // N_ACTIONS
HALF = SEG // 2
W_PER_SEG = (NC * NS) // N_ACTIONS
CHUNK = HALF // W_PER_SEG
NBUF = 4
SUB = CHUNK // NBUF
ACCS = 8
STEPS = SUB // (ACCS * L)


def _policy_body(probs_hbm, out_hbm, chunk_v, acc_v, part_v, partials_hbm,
                 *sems):
    c = lax.axis_index("c")
    sid = lax.axis_index("s")
    seg = c * (N_ACTIONS // NC) + sid // W_PER_SEG
    base = seg * SEG + (sid % W_PER_SEG) * CHUNK

    copies = [
        pltpu.async_copy(
            probs_hbm.at[pl.ds(base + b * SUB, SUB)], chunk_v.at[b], sems[b]
        )
        for b in range(NBUF)
    ]

    zero = jnp.zeros((L,), jnp.float32)
    accs = (zero,) * ACCS
    for b in range(NBUF):
        copies[b].wait()

        def body(i, accs, b=b):
            off = i * (ACCS * L)
            return tuple(
                a + chunk_v[b, pl.ds(off + k * L, L)]
                for k, a in enumerate(accs)
            )

        accs = lax.fori_loop(0, STEPS, body, accs)
    acc = accs[0]
    for a in accs[1:]:
        acc = acc + a
    acc_v[...] = acc
    pltpu.sync_copy(acc_v, partials_hbm.at[c, sid])
    plsc.subcore_barrier()

    @pl.when(sid % W_PER_SEG == 0)
    def _():
        pltpu.sync_copy(partials_hbm.at[c, pl.ds(sid, W_PER_SEG)], part_v)
        tot = part_v[0]
        for k in range(1, W_PER_SEG):
            tot = tot + part_v[k]
        total = tot[0]
        for k in range(1, L):
            total = total + tot[k]
        lane = lax.iota(jnp.int32, L)
        acc_v[...] = jnp.where(lane == 0, total, jnp.float32(0.0))
        pltpu.sync_copy(acc_v, out_hbm.at[seg])


_policy_sc = functools.partial(
    pl.kernel,
    out_type=jax.ShapeDtypeStruct((N_ACTIONS, L), jnp.float32),
    mesh=plsc.VectorSubcoreMesh(
        core_axis_name="c", subcore_axis_name="s", num_cores=NC, num_subcores=NS
    ),
    scratch_types=[
        pltpu.VMEM((NBUF, SUB), jnp.float32),
        pltpu.VMEM((L,), jnp.float32),
        pltpu.VMEM((W_PER_SEG, L), jnp.float32),
        pltpu.HBM((NC, NS, L), jnp.float32),
    ] + [pltpu.SemaphoreType.DMA] * NBUF,
)(_policy_body)


TC_SL = HALF // 128


def _tc_body(x_ref, out_ref):
    i = pl.program_id(0)
    s = jnp.sum(x_ref[...])
    out_ref[pl.ds(i, 1), :] = jnp.full((1, 128), s, jnp.float32)


_tc_half = pl.pallas_call(
    _tc_body,
    grid=(N_ACTIONS,),
    in_specs=[pl.BlockSpec((1, TC_SL, 128), lambda i: (2 * i + 1, 0, 0))],
    out_specs=pl.BlockSpec((8, 128), lambda i: (0, 0)),
    out_shape=jax.ShapeDtypeStruct((8, 128), jnp.float32),
)


def kernel(probs):
    flat = probs.reshape(-1)
    sc16 = _policy_sc(flat)
    tc8 = _tc_half(flat.reshape(2 * N_ACTIONS, TC_SL, 128))
    return sc16[:, 0] + tc8[:N_ACTIONS, 0]


# speedup vs baseline: 1.0446x; 1.0003x over previous
import functools

import jax
import jax.numpy as jnp
from jax import lax
from jax.experimental import pallas as pl
from jax.experimental.pallas import tpu as pltpu
from jax.experimental.pallas import tpu_sc as plsc

N = 1 << 20
N_ACTIONS = 4
NC = 2
NS = 16
L = 16
SEG = N // N_ACTIONS
HALF = SEG // 2
W_PER_SEG = (NC * NS) // N_ACTIONS
CHUNK = HALF // W_PER_SEG
NBUF = 4
SUB = CHUNK // NBUF
ACCS = 8
STEPS = SUB // (ACCS * L)


def _policy_body(probs_hbm, out_hbm, chunk_v, acc_v, part_v, partials_hbm,
                 *sems):
    c = lax.axis_index("c")
    sid = lax.axis_index("s")
    seg = c * (N_ACTIONS // NC) + sid // W_PER_SEG
    base = seg * SEG + (sid % W_PER_SEG) * CHUNK

    copies = [
        pltpu.async_copy(
            probs_hbm.at[pl.ds(base + b * SUB, SUB)], chunk_v.at[b], sems[b]
        )
        for b in range(NBUF)
    ]

    zero = jnp.zeros((L,), jnp.float32)
    accs = (zero,) * ACCS
    for b in range(NBUF):
        copies[b].wait()

        def body(i, accs, b=b):
            off = i * (ACCS * L)
            return tuple(
                a + chunk_v[b, pl.ds(off + k * L, L)]
                for k, a in enumerate(accs)
            )

        accs = lax.fori_loop(0, STEPS, body, accs)
    acc = accs[0]
    for a in accs[1:]:
        acc = acc + a
    acc_v[...] = acc
    pltpu.sync_copy(acc_v, partials_hbm.at[c, sid])
    plsc.subcore_barrier()

    @pl.when(sid % W_PER_SEG == 0)
    def _():
        pltpu.sync_copy(partials_hbm.at[c, pl.ds(sid, W_PER_SEG)], part_v)
        tot = part_v[0]
        for k in range(1, W_PER_SEG):
            tot = tot + part_v[k]
        total = tot[0]
        for k in range(1, L):
            total = total + tot[k]
        lane = lax.iota(jnp.int32, L)
        acc_v[...] = jnp.where(lane == 0, total, jnp.float32(0.0))
        pltpu.sync_copy(acc_v, out_hbm.at[seg])


_policy_sc = functools.partial(
    pl.kernel,
    out_type=jax.ShapeDtypeStruct((N_ACTIONS, L), jnp.float32),
    mesh=plsc.VectorSubcoreMesh(
        core_axis_name="c", subcore_axis_name="s", num_cores=NC, num_subcores=NS
    ),
    scratch_types=[
        pltpu.VMEM((NBUF, SUB), jnp.float32),
        pltpu.VMEM((L,), jnp.float32),
        pltpu.VMEM((W_PER_SEG, L), jnp.float32),
        pltpu.HBM((NC, NS, L), jnp.float32),
    ] + [pltpu.SemaphoreType.DMA] * NBUF,
)(_policy_body)


TC_SL = HALF // 128


def _tc_body(x_ref, out_ref):
    i = pl.program_id(0)
    out_ref[pl.ds(i, 1), :] = jnp.sum(x_ref[0], axis=0, keepdims=True)


_tc_half = pl.pallas_call(
    _tc_body,
    grid=(N_ACTIONS,),
    in_specs=[pl.BlockSpec((1, TC_SL, 128), lambda i: (2 * i + 1, 0, 0))],
    out_specs=pl.BlockSpec((8, 128), lambda i: (0, 0)),
    out_shape=jax.ShapeDtypeStruct((8, 128), jnp.float32),
)


def kernel(probs):
    flat = probs.reshape(-1)
    sc16 = _policy_sc(flat)
    tc8 = _tc_half(flat.reshape(2 * N_ACTIONS, TC_SL, 128))
    return sc16[:, 0] + jnp.sum(tc8[:N_ACTIONS], axis=1)
